# baseline (speedup 1.0000x reference)
import functools

import jax
import jax.numpy as jnp
from jax import lax
from jax.experimental import pallas as pl
from jax.experimental.pallas import tpu as pltpu
from jax.experimental.pallas import tpu_sc as plsc

_B, _S, _D = 4, 4096, 2048
_T = 3
_NW = 32
_RPW = (_B * _S) // _NW
_CH = 32


def _sc_concat(x_hbm, emb_hbm, out_hbm, buf, ebuf, idx, sem):
    c = lax.axis_index("c")
    s = lax.axis_index("s")
    wid = s * 2 + c
    wpb = _NW // _B
    b = wid // wpb
    r0 = (wid % wpb) * _RPW

    @pl.when(wid % wpb == 0)
    def _():
        pltpu.sync_copy(emb_hbm, ebuf.at[pl.ds(0, _T)])
        pltpu.sync_copy(ebuf.at[pl.ds(0, _T)], out_hbm.at[b, pl.ds(0, _T)])

    lanes = lax.iota(jnp.int32, 16)

    def body(i, carry):
        r = r0 + i * _CH
        pltpu.sync_copy(x_hbm.at[b, pl.ds(r, _CH)], buf)
        for k in range(_CH // 16):
            idx[pl.ds(16 * k, 16)] = (r + _T + 16 * k) + lanes
        pltpu.async_copy(buf, out_hbm.at[b].at[idx], sem).wait()
        return carry

    lax.fori_loop(0, _RPW // _CH, body, 0)


def kernel(x, embedding):
    mesh = plsc.VectorSubcoreMesh(core_axis_name="c", subcore_axis_name="s")
    k = functools.partial(
        pl.kernel,
        mesh=mesh,
        out_type=jax.ShapeDtypeStruct((_B, _S + _T, _D), x.dtype),
        scratch_types=[
            pltpu.VMEM((_CH, _D), jnp.float32),
            pltpu.VMEM((8, _D), jnp.float32),
            pltpu.VMEM((_CH,), jnp.int32),
            pltpu.SemaphoreType.DMA,
        ],
    )(_sc_concat)
    return k(x, embedding)


# speedup vs baseline: 1.0211x; 1.0211x over previous
import functools

import jax
import jax.numpy as jnp
from jax import lax
from jax.experimental import pallas as pl
from jax.experimental.pallas import tpu as pltpu
from jax.experimental.pallas import tpu_sc as plsc

_B, _S, _D = 4, 4096, 2048
_T = 3
_NW = 32
_RPW = (_B * _S) // _NW
_CH = 16
_NCH = _RPW // _CH


def _sc_concat(x_hbm, emb_hbm, out_hbm, buf0, buf1, ebuf, sem_r, sem_w):
    c = lax.axis_index("c")
    s = lax.axis_index("s")
    wid = s * 2 + c
    wpb = _NW // _B
    b = wid // wpb
    r0 = (wid % wpb) * _RPW
    bufs = (buf0, buf1)
    lanes = lax.iota(jnp.int32, _CH)

    @pl.when(wid % wpb == 0)
    def _():
        pltpu.sync_copy(emb_hbm, ebuf.at[pl.ds(0, _T)])
        pltpu.sync_copy(ebuf.at[pl.ds(0, _T)], out_hbm.at[b, pl.ds(0, _T)])

    def gather(i, buf):
        pltpu.async_copy(x_hbm.at[b, pl.ds(r0 + i * _CH, _CH)], buf, sem_r)

    def scatter(i, buf):
        idx = (r0 + i * _CH + _T) + lanes
        pltpu.async_copy(buf, out_hbm.at[b].at[idx], sem_w)

    def wait_chunk(buf, sem):
        pltpu.make_async_copy(x_hbm.at[b, pl.ds(0, _CH)], buf, sem).wait()

    gather(0, bufs[0])

    def body(i2, carry):
        for j in range(2):
            i = i2 * 2 + j
            cur = bufs[j]
            nxt = bufs[1 - j]
            wait_chunk(cur, sem_r)
            scatter(i, cur)

            @pl.when(i >= 1)
            def _():
                wait_chunk(nxt, sem_w)

            @pl.when(i + 1 < _NCH)
            def _():
                gather(i + 1, nxt)
        return carry

    lax.fori_loop(0, _NCH // 2, body, 0)
    wait_chunk(bufs[(_NCH - 1) % 2], sem_w)


def kernel(x, embedding):
    mesh = plsc.VectorSubcoreMesh(core_axis_name="c", subcore_axis_name="s")
    k = functools.partial(
        pl.kernel,
        mesh=mesh,
        out_type=jax.ShapeDtypeStruct((_B, _S + _T, _D), x.dtype),
        scratch_types=[
            pltpu.VMEM((_CH, _D), jnp.float32),
            pltpu.VMEM((_CH, _D), jnp.float32),
            pltpu.VMEM((8, _D), jnp.float32),
            pltpu.SemaphoreType.DMA,
            pltpu.SemaphoreType.DMA,
        ],
    )(_sc_concat)
    return k(x, embedding)
